# Initial kernel scaffold; baseline (speedup 1.0000x reference)
import jax
import jax.numpy as jnp
from jax.experimental import pallas as pl


def kernel(x, Wp, bp, Eemb, W1, b1, W2, b2):
    raise NotImplementedError("write your pallas kernel here")


# speedup vs baseline: 7.0377x; 7.0377x over previous
import functools

import jax
import jax.numpy as jnp
from jax import lax
from jax.experimental import pallas as pl
from jax.experimental.pallas import tpu as pltpu
from jax.experimental.pallas import tpu_sc as plsc

TOKENS = 8192
DIM_IN = 768
DIM_HID = 768
DIM_OUT = 768
NE = 64
TAU = 5.0

BLK = 256
NBLK = TOKENS // BLK
TILE = 128
RPAD = 16384
NTILE = RPAD // TILE

NC = 2
NS = 16
NW = NC * NS
TPW = TOKENS // NW
CH = 64
LANES = 16


def _router_body(x_ref, Wp_ref, bp_ref, Ee_ref,
                 p_ref, sel_ref, w_ref, rank_ref, offs_ref, meta_ref,
                 counts):
    i = pl.program_id(0)

    @pl.when(i == 0)
    def _():
        counts[...] = jnp.zeros_like(counts)

    xb = x_ref[...]
    h = jnp.dot(xb, Wp_ref[...], preferred_element_type=jnp.float32)
    h = h + bp_ref[0:1, :]
    logits = jnp.dot(h, Ee_ref[...], preferred_element_type=jnp.float32)

    s = logits / TAU
    m = jnp.maximum(s, 0.0)
    eb = jnp.exp(s - m)
    ea = jnp.exp(-m)
    p = eb / (ea + eb)
    p_ref[...] = p

    pmax = jnp.max(p, axis=1, keepdims=True)
    eids = lax.broadcasted_iota(jnp.int32, (BLK, NE), 1)
    selc = jnp.min(jnp.where(p >= pmax, eids, NE), axis=1)
    v = pmax[:, 0]
    w = v / (v + 1e-10)

    onehot = (eids == selc[:, None]).astype(jnp.float32)
    r_i = lax.broadcasted_iota(jnp.int32, (BLK, BLK), 0)
    c_i = lax.broadcasted_iota(jnp.int32, (BLK, BLK), 1)
    tril = (c_i < r_i).astype(jnp.float32)
    pref = jnp.dot(tril, onehot, preferred_element_type=jnp.float32)
    rank_in = jnp.sum(pref * onehot, axis=1)
    carry = jnp.sum(counts[...] * onehot, axis=1)
    rank = (rank_in + carry).astype(jnp.int32)
    counts[...] = counts[...] + jnp.sum(onehot, axis=0, keepdims=True)

    sel_ref[...] = selc.reshape(1, 1, BLK)
    w_ref[...] = w.reshape(1, 1, BLK)
    rank_ref[...] = rank.reshape(1, 1, BLK)

    @pl.when(i == NBLK - 1)
    def _():
        cnt = counts[...]
        cpad = jnp.floor((cnt + (TILE - 1)) / TILE) * TILE
        e_r = lax.broadcasted_iota(jnp.int32, (NE, NE), 0)
        e_c = lax.broadcasted_iota(jnp.int32, (NE, NE), 1)
        supper = (e_r < e_c).astype(jnp.float32)
        offs = jnp.dot(cpad, supper, preferred_element_type=jnp.float32)
        offs_i = offs.astype(jnp.int32)
        offs_ref[...] = jnp.broadcast_to(offs_i, (8, NE))
        starts = lax.broadcasted_iota(jnp.int32, (NTILE, NE), 0) * TILE
        te = jnp.sum((offs_i <= starts).astype(jnp.int32), axis=1) - 1
        nvalid = (jnp.sum(cpad) / TILE).astype(jnp.int32)
        meta_ref[...] = jnp.concatenate(
            [jnp.broadcast_to(te.reshape(1, NTILE), (8, NTILE)),
             jnp.full((8, NTILE), nvalid, jnp.int32)], axis=1)


def _router_call(x, Wp, bp, Eemb):
    return pl.pallas_call(
        _router_body,
        grid=(NBLK,),
        in_specs=[
            pl.BlockSpec((BLK, DIM_IN), lambda i: (i, 0)),
            pl.BlockSpec((DIM_IN, NE), lambda i: (0, 0)),
            pl.BlockSpec((8, NE), lambda i: (0, 0)),
            pl.BlockSpec((NE, NE), lambda i: (0, 0)),
        ],
        out_specs=[
            pl.BlockSpec((BLK, NE), lambda i: (i, 0)),
            pl.BlockSpec((1, 1, BLK), lambda i: (i, 0, 0)),
            pl.BlockSpec((1, 1, BLK), lambda i: (i, 0, 0)),
            pl.BlockSpec((1, 1, BLK), lambda i: (i, 0, 0)),
            pl.BlockSpec((8, NE), lambda i: (0, 0)),
            pl.BlockSpec((8, 2 * NTILE), lambda i: (0, 0)),
        ],
        out_shape=[
            jax.ShapeDtypeStruct((TOKENS, NE), jnp.float32),
            jax.ShapeDtypeStruct((NBLK, 1, BLK), jnp.int32),
            jax.ShapeDtypeStruct((NBLK, 1, BLK), jnp.float32),
            jax.ShapeDtypeStruct((NBLK, 1, BLK), jnp.int32),
            jax.ShapeDtypeStruct((8, NE), jnp.int32),
            jax.ShapeDtypeStruct((8, 2 * NTILE), jnp.int32),
        ],
        scratch_shapes=[pltpu.VMEM((1, NE), jnp.float32)],
    )(x, Wp, bp, Eemb)


def _expert_body(meta_ref, xs_ref, W1_ref, b1_ref, W2_ref, b2_ref, ys_ref):
    i = pl.program_id(0)
    nvalid = meta_ref[NTILE]

    @pl.when(i < nvalid)
    def _():
        xb = xs_ref[...]
        h = jnp.dot(xb, W1_ref[0], preferred_element_type=jnp.float32)
        h = jnp.maximum(h + b1_ref[0], 0.0)
        y = jnp.dot(h, W2_ref[0], preferred_element_type=jnp.float32)
        ys_ref[...] = y + b2_ref[0]


def _expert_call(meta, xs, W1, b1, W2, b2):
    grid_spec = pltpu.PrefetchScalarGridSpec(
        num_scalar_prefetch=1,
        grid=(NTILE,),
        in_specs=[
            pl.BlockSpec((TILE, DIM_IN), lambda i, m: (i, 0)),
            pl.BlockSpec((1, DIM_IN, DIM_HID), lambda i, m: (m[i], 0, 0)),
            pl.BlockSpec((1, 1, DIM_HID), lambda i, m: (m[i], 0, 0)),
            pl.BlockSpec((1, DIM_HID, DIM_OUT), lambda i, m: (m[i], 0, 0)),
            pl.BlockSpec((1, 1, DIM_OUT), lambda i, m: (m[i], 0, 0)),
        ],
        out_specs=pl.BlockSpec((TILE, DIM_OUT), lambda i, m: (i, 0)),
    )
    return pl.pallas_call(
        _expert_body,
        grid_spec=grid_spec,
        out_shape=jax.ShapeDtypeStruct((RPAD, DIM_OUT), jnp.float32),
    )(meta, xs, W1, b1.reshape(NE, 1, DIM_HID), W2, b2.reshape(NE, 1, DIM_OUT))


@functools.cache
def _sc_mesh():
    return plsc.VectorSubcoreMesh(core_axis_name="c", subcore_axis_name="s",
                                  num_cores=NC)


@functools.cache
def _dispatch_call():
    @functools.partial(
        pl.kernel,
        out_type=(jax.ShapeDtypeStruct((RPAD, DIM_IN), jnp.float32),
                  jax.ShapeDtypeStruct((TOKENS,), jnp.int32)),
        mesh=_sc_mesh(),
        compiler_params=pltpu.CompilerParams(needs_layout_passes=False),
        scratch_types=[
            pltpu.VMEM((CH, DIM_IN), jnp.float32),
            pltpu.VMEM((CH,), jnp.int32),
            pltpu.VMEM((CH,), jnp.int32),
            pltpu.VMEM((NE,), jnp.int32),
            pltpu.VMEM((CH,), jnp.int32),
            pltpu.SemaphoreType.DMA,
        ],
    )
    def _dispatch(x_hbm, sel_hbm, rank_hbm, offs_hbm, xs_hbm, dst_hbm,
                  xv, selv, rankv, offsv, dstv, sem):
        wid = lax.axis_index("s") * NC + lax.axis_index("c")
        base = wid * TPW
        pltpu.sync_copy(offs_hbm, offsv)
        for c in range(TPW // CH):
            off = base + c * CH
            pltpu.sync_copy(x_hbm.at[pl.ds(off, CH)], xv)
            pltpu.sync_copy(sel_hbm.at[pl.ds(off, CH)], selv)
            pltpu.sync_copy(rank_hbm.at[pl.ds(off, CH)], rankv)
            for g in range(CH // LANES):
                s16 = selv[pl.ds(g * LANES, LANES)]
                r16 = rankv[pl.ds(g * LANES, LANES)]
                o16 = plsc.load_gather(offsv, [s16])
                dstv[pl.ds(g * LANES, LANES)] = o16 + r16
            pltpu.async_copy(xv, xs_hbm.at[dstv], sem).wait()
            pltpu.sync_copy(dstv, dst_hbm.at[pl.ds(off, CH)])

    return _dispatch


@functools.cache
def _combine_call():
    @functools.partial(
        pl.kernel,
        out_type=jax.ShapeDtypeStruct((TOKENS, DIM_OUT), jnp.float32),
        mesh=_sc_mesh(),
        compiler_params=pltpu.CompilerParams(needs_layout_passes=False),
        scratch_types=[
            pltpu.VMEM((CH, DIM_OUT), jnp.float32),
            pltpu.VMEM((CH,), jnp.int32),
            pltpu.VMEM((CH,), jnp.float32),
            pltpu.SemaphoreType.DMA,
        ],
    )
    def _combine(ys_hbm, dst_hbm, w_hbm, out_hbm, yv, dstv, wv, sem):
        wid = lax.axis_index("s") * NC + lax.axis_index("c")
        base = wid * TPW
        for c in range(TPW // CH):
            off = base + c * CH
            pltpu.sync_copy(dst_hbm.at[pl.ds(off, CH)], dstv)
            pltpu.sync_copy(w_hbm.at[pl.ds(off, CH)], wv)
            pltpu.async_copy(ys_hbm.at[dstv], yv, sem).wait()

            def body(t, carry):
                wb = plsc.load_gather(wv, [jnp.full((LANES,), 0, jnp.int32) + t])
                for k in range(DIM_OUT // LANES):
                    yv[t, pl.ds(k * LANES, LANES)] = (
                        yv[t, pl.ds(k * LANES, LANES)] * wb)
                return carry

            lax.fori_loop(0, CH, body, 0)
            pltpu.sync_copy(yv, out_hbm.at[pl.ds(off, CH)])

    return _combine


def kernel(x, Wp, bp, Eemb, W1, b1, W2, b2):
    bp8 = jnp.broadcast_to(bp.reshape(1, NE), (8, NE))
    p_open, sel3, w3, rank3, offs8, meta8 = _router_call(x, Wp, bp8, Eemb)
    sel = sel3.reshape(TOKENS)
    w = w3.reshape(TOKENS)
    rank = rank3.reshape(TOKENS)
    offs = offs8[0]
    meta = meta8[0]
    xs, dst = _dispatch_call()(x, sel, rank, offs)
    ys = _expert_call(meta, xs, W1, b1, W2, b2)
    out = _combine_call()(ys, dst, w)
    return out, jnp.zeros((), x.dtype), p_open
